# Initial kernel scaffold; baseline (speedup 1.0000x reference)
import jax
import jax.numpy as jnp
from jax.experimental import pallas as pl


def kernel(hidden_states, top_k_index, top_k_weights, gate_up_proj, down_proj):
    raise NotImplementedError("write your pallas kernel here")


# speedup vs baseline: 1.5996x; 1.5996x over previous
import functools

import jax
import jax.numpy as jnp
from jax.experimental import pallas as pl
from jax.experimental.pallas import tpu as pltpu

_E = 8
_H = 1024
_I = 512
_T = 2048
_K = 2
_TB = 256


def _dense_moe_kernel(idx_ref, w_ref, x_ref, gup_ref, down_ref, out_ref):
    x = x_ref[...]
    idx = idx_ref[...]
    w = w_ref[...]
    acc = jnp.zeros(out_ref.shape, jnp.float32)
    for e in range(_E):
        gu = jax.lax.dot_general(
            x, gup_ref[e],
            (((1,), (1,)), ((), ())),
            preferred_element_type=jnp.float32,
        )
        gate = gu[:, :_I]
        up = gu[:, _I:]
        h = (gate * jax.nn.sigmoid(gate) * up).astype(jnp.bfloat16)
        y = jax.lax.dot_general(
            h, down_ref[e],
            (((1,), (1,)), ((), ())),
            preferred_element_type=jnp.float32,
        )
        c = jnp.sum(jnp.where(idx == e, w, 0.0), axis=1, keepdims=True)
        acc = acc + y * c
    out_ref[...] = acc


def kernel(hidden_states, top_k_index, top_k_weights, gate_up_proj, down_proj):
    x16 = hidden_states.astype(jnp.bfloat16)
    gup16 = gate_up_proj.astype(jnp.bfloat16)
    down16 = down_proj.astype(jnp.bfloat16)
    idx32 = top_k_index.astype(jnp.int32)

    return pl.pallas_call(
        _dense_moe_kernel,
        grid=(_T // _TB,),
        in_specs=[
            pl.BlockSpec((_TB, _K), lambda i: (i, 0)),
            pl.BlockSpec((_TB, _K), lambda i: (i, 0)),
            pl.BlockSpec((_TB, _H), lambda i: (i, 0)),
            pl.BlockSpec((_E, 2 * _I, _H), lambda i: (0, 0, 0)),
            pl.BlockSpec((_E, _H, _I), lambda i: (0, 0, 0)),
        ],
        out_specs=pl.BlockSpec((_TB, _H), lambda i: (i, 0)),
        out_shape=jax.ShapeDtypeStruct((_T, _H), jnp.float32),
        compiler_params=pltpu.CompilerParams(
            vmem_limit_bytes=100 * 1024 * 1024,
        ),
    )(idx32, top_k_weights, x16, gup16, down16)
